# Initial kernel scaffold; baseline (speedup 1.0000x reference)
import functools

import jax
import jax.numpy as jnp
from jax import lax
from jax.experimental import pallas as pl
from jax.experimental.pallas import tpu as pltpu
from jax.experimental.pallas import tpu_sc as plsc

_VOCAB = 1_000_000
_EMB = 32
_PROJ_BLK = 40_000


def _proj_body(table_ref, w_ref, b_ref, out_ref):
    out_ref[...] = (
        jnp.dot(table_ref[...], w_ref[...], preferred_element_type=jnp.float32)
        + b_ref[0, 0]
    )


def _project(table, W, b):
    grid = _VOCAB // _PROJ_BLK
    return pl.pallas_call(
        _proj_body,
        grid=(grid,),
        in_specs=[
            pl.BlockSpec((_PROJ_BLK, _EMB), lambda i: (i, 0)),
            pl.BlockSpec((_EMB, 1), lambda i: (0, 0)),
            pl.BlockSpec(memory_space=pltpu.SMEM),
        ],
        out_specs=pl.BlockSpec((_PROJ_BLK, 1), lambda i: (i, 0)),
        out_shape=jax.ShapeDtypeStruct((_VOCAB, 1), jnp.float32),
    )(table, W, b.reshape(1, 1))


@functools.cache
def _gather_fn(total):
    info = plsc.get_sparse_core_info()
    nc, ns = info.num_cores, info.num_subcores
    nw = nc * ns
    per_w = total // nw
    assert per_w * nw == total and per_w % 8 == 0
    chunk = 12_800
    nchunk = per_w // chunk
    assert nchunk * chunk == per_w

    mesh = plsc.VectorSubcoreMesh(core_axis_name="c", subcore_axis_name="s")

    @functools.partial(
        pl.kernel,
        out_type=jax.ShapeDtypeStruct((total,), jnp.float32),
        mesh=mesh,
        scratch_types=[
            pltpu.VMEM((chunk,), jnp.int32),
            pltpu.VMEM((chunk,), jnp.float32),
            pltpu.SemaphoreType.DMA,
        ],
    )
    def body(tv_hbm, idx_hbm, out_hbm, idx_v, rows_v, sem):
        wid = lax.axis_index("s") * nc + lax.axis_index("c")
        base = wid * per_w
        for c in range(nchunk):
            off = base + c * chunk
            pltpu.sync_copy(idx_hbm.at[pl.ds(off, chunk)], idx_v)
            pltpu.async_copy(tv_hbm.at[idx_v], rows_v, sem).wait()
            pltpu.sync_copy(rows_v, out_hbm.at[pl.ds(off, chunk)])

    return body


def kernel(batch, table, W, b):
    B, L = batch.shape
    tv = _project(table, W, b).reshape(_VOCAB)
    flat = _gather_fn(B * L)(tv, batch.reshape(-1))
    return flat.reshape(B, L, 1)


# speedup vs baseline: 11.9955x; 11.9955x over previous
import functools

import jax
import jax.numpy as jnp
from jax import lax
from jax.experimental import pallas as pl
from jax.experimental.pallas import tpu as pltpu
from jax.experimental.pallas import tpu_sc as plsc

_VOCAB = 1_000_000
_EMB = 32
_PROJ_BLK = 8_000


def _proj_body(table_ref, w_ref, b_ref, out_ref):
    out_ref[...] = (
        jnp.dot(table_ref[...], w_ref[...], preferred_element_type=jnp.float32)
        + b_ref[0, 0]
    )


def _project(table, W, b):
    grid = _VOCAB // _PROJ_BLK
    return pl.pallas_call(
        _proj_body,
        grid=(grid,),
        in_specs=[
            pl.BlockSpec((_PROJ_BLK, _EMB), lambda i: (i, 0)),
            pl.BlockSpec((_EMB, 1), lambda i: (0, 0)),
            pl.BlockSpec(memory_space=pltpu.SMEM),
        ],
        out_specs=pl.BlockSpec((_PROJ_BLK, 1), lambda i: (i, 0)),
        out_shape=jax.ShapeDtypeStruct((_VOCAB, 1), jnp.float32),
    )(table, W, b.reshape(1, 1))


@functools.cache
def _gather_fn(total):
    info = plsc.get_sparse_core_info()
    nc, ns = info.num_cores, info.num_subcores
    nw = nc * ns
    per_w = total // nw
    assert per_w * nw == total and per_w % 8 == 0
    chunk = 12_800
    nchunk = per_w // chunk
    assert nchunk * chunk == per_w

    mesh = plsc.VectorSubcoreMesh(core_axis_name="c", subcore_axis_name="s")

    @functools.partial(
        pl.kernel,
        out_type=jax.ShapeDtypeStruct((total,), jnp.float32),
        mesh=mesh,
        scratch_types=[
            pltpu.VMEM((chunk,), jnp.int32),
            pltpu.VMEM((chunk,), jnp.float32),
            pltpu.SemaphoreType.DMA,
        ],
    )
    def body(tv_hbm, idx_hbm, out_hbm, idx_v, rows_v, sem):
        wid = lax.axis_index("s") * nc + lax.axis_index("c")
        base = wid * per_w
        for c in range(nchunk):
            off = base + c * chunk
            pltpu.sync_copy(idx_hbm.at[pl.ds(off, chunk)], idx_v)
            pltpu.async_copy(tv_hbm.at[idx_v], rows_v, sem).wait()
            pltpu.sync_copy(rows_v, out_hbm.at[pl.ds(off, chunk)])

    return body


def kernel(batch, table, W, b):
    B, L = batch.shape
    tv = _project(table, W, b).reshape(_VOCAB)
    flat = _gather_fn(B * L)(tv, batch.reshape(-1))
    return flat.reshape(B, L, 1)


# speedup vs baseline: 16.1350x; 1.3451x over previous
import functools

import jax
import jax.numpy as jnp
from jax import lax
from jax.experimental import pallas as pl
from jax.experimental.pallas import tpu as pltpu
from jax.experimental.pallas import tpu_sc as plsc

_VOCAB = 1_000_000
_EMB = 32
_PROJ_BLK = 8_192
_PROJ_GRID = 123
_OUT_ROWS = _PROJ_BLK // 128


def _proj_body(table_ref, w_ref, b_ref, out_ref):
    y = (
        jnp.dot(table_ref[...], w_ref[...], preferred_element_type=jnp.float32)
        + b_ref[0, 0]
    )
    out_ref[...] = y.reshape(_OUT_ROWS, 128)


def _project(table, W, b):
    return pl.pallas_call(
        _proj_body,
        grid=(_PROJ_GRID,),
        in_specs=[
            pl.BlockSpec((_PROJ_BLK, _EMB), lambda i: (i, 0)),
            pl.BlockSpec((_EMB, 1), lambda i: (0, 0)),
            pl.BlockSpec(memory_space=pltpu.SMEM),
        ],
        out_specs=pl.BlockSpec((_OUT_ROWS, 128), lambda i: (i, 0)),
        out_shape=jax.ShapeDtypeStruct((_PROJ_GRID * _OUT_ROWS, 128), jnp.float32),
    )(table, W, b.reshape(1, 1))


@functools.cache
def _gather_fn(total):
    info = plsc.get_sparse_core_info()
    nc, ns = info.num_cores, info.num_subcores
    nw = nc * ns
    per_w = total // nw
    assert per_w * nw == total and per_w % 8 == 0
    chunk = 12_800
    nchunk = per_w // chunk
    assert nchunk * chunk == per_w

    mesh = plsc.VectorSubcoreMesh(core_axis_name="c", subcore_axis_name="s")

    @functools.partial(
        pl.kernel,
        out_type=jax.ShapeDtypeStruct((total,), jnp.float32),
        mesh=mesh,
        scratch_types=[
            pltpu.VMEM((chunk,), jnp.int32),
            pltpu.VMEM((chunk,), jnp.float32),
            pltpu.SemaphoreType.DMA,
        ],
    )
    def body(tv_hbm, idx_hbm, out_hbm, idx_v, rows_v, sem):
        wid = lax.axis_index("s") * nc + lax.axis_index("c")
        base = wid * per_w
        for c in range(nchunk):
            off = base + c * chunk
            pltpu.sync_copy(idx_hbm.at[pl.ds(off, chunk)], idx_v)
            pltpu.async_copy(tv_hbm.at[idx_v], rows_v, sem).wait()
            pltpu.sync_copy(rows_v, out_hbm.at[pl.ds(off, chunk)])

    return body


def kernel(batch, table, W, b):
    B, L = batch.shape
    tv = _project(table, W, b).reshape(-1)
    flat = _gather_fn(B * L)(tv, batch.reshape(-1))
    return flat.reshape(B, L, 1)


# speedup vs baseline: 51.5119x; 3.1926x over previous
import functools

import jax
import jax.numpy as jnp
from jax import lax
from jax.experimental import pallas as pl
from jax.experimental.pallas import tpu as pltpu
from jax.experimental.pallas import tpu_sc as plsc

_VOCAB = 1_000_000
_EMB = 32
def _proj_body(tt_ref, w_ref, b_ref, out_ref):
    y = jnp.sum(tt_ref[...] * w_ref[...], axis=0, keepdims=True) + b_ref[0, 0]
    out_ref[...] = y.reshape(1, 1, _PROJ_COLS)


_PROJ_COLS = 65_536
_PROJ_GRID = 16


def _project(table, W, b):
    return pl.pallas_call(
        _proj_body,
        grid=(_PROJ_GRID,),
        in_specs=[
            pl.BlockSpec((_EMB, _PROJ_COLS), lambda i: (0, i)),
            pl.BlockSpec((_EMB, 1), lambda i: (0, 0)),
            pl.BlockSpec(memory_space=pltpu.SMEM),
        ],
        out_specs=pl.BlockSpec((1, 1, _PROJ_COLS), lambda i: (i, 0, 0)),
        out_shape=jax.ShapeDtypeStruct((_PROJ_GRID, 1, _PROJ_COLS), jnp.float32),
    )(table.T, W, b.reshape(1, 1))


@functools.cache
def _gather_fn(total):
    info = plsc.get_sparse_core_info()
    nc, ns = info.num_cores, info.num_subcores
    nw = nc * ns
    per_w = total // nw
    assert per_w * nw == total and per_w % 8 == 0
    chunk = 5_120
    nchunk = per_w // chunk
    assert nchunk * chunk == per_w
    nbuf = 3

    mesh = plsc.VectorSubcoreMesh(core_axis_name="c", subcore_axis_name="s")

    @functools.partial(
        pl.kernel,
        out_type=jax.ShapeDtypeStruct((total,), jnp.float32),
        mesh=mesh,
        scratch_types=(
            [pltpu.VMEM((chunk,), jnp.int32) for _ in range(nbuf)]
            + [pltpu.VMEM((chunk,), jnp.float32) for _ in range(nbuf)]
            + [pltpu.SemaphoreType.DMA for _ in range(3 * nbuf)]
        ),
    )
    def body(tv_hbm, idx_hbm, out_hbm, *scratch):
        idx_v = scratch[0:nbuf]
        rows_v = scratch[nbuf : 2 * nbuf]
        s_i = scratch[2 * nbuf : 3 * nbuf]
        s_g = scratch[3 * nbuf : 4 * nbuf]
        s_o = scratch[4 * nbuf : 5 * nbuf]
        wid = lax.axis_index("s") * nc + lax.axis_index("c")
        base = wid * per_w

        def off(c):
            return base + c * chunk

        h_i, h_g, h_o = {}, {}, {}
        for p in range(min(2, nchunk)):
            h_i[p] = pltpu.async_copy(
                idx_hbm.at[pl.ds(off(p), chunk)], idx_v[p % nbuf], s_i[p % nbuf]
            )
        for c in range(nchunk):
            s = c % nbuf
            if c - nbuf >= 0:
                h_o[c - nbuf].wait()
            h_i[c].wait()
            h_g[c] = pltpu.async_copy(tv_hbm.at[idx_v[s]], rows_v[s], s_g[s])
            if c >= 1:
                h_g[c - 1].wait()
                h_o[c - 1] = pltpu.async_copy(
                    rows_v[(c - 1) % nbuf],
                    out_hbm.at[pl.ds(off(c - 1), chunk)],
                    s_o[(c - 1) % nbuf],
                )
            if c + 2 < nchunk:
                h_i[c + 2] = pltpu.async_copy(
                    idx_hbm.at[pl.ds(off(c + 2), chunk)],
                    idx_v[(c + 2) % nbuf],
                    s_i[(c + 2) % nbuf],
                )
        h_g[nchunk - 1].wait()
        h_o[nchunk - 1] = pltpu.async_copy(
            rows_v[(nchunk - 1) % nbuf],
            out_hbm.at[pl.ds(off(nchunk - 1), chunk)],
            s_o[(nchunk - 1) % nbuf],
        )
        for c in range(max(0, nchunk - nbuf), nchunk):
            h_o[c].wait()

    return body


def kernel(batch, table, W, b):
    B, L = batch.shape
    tv = _project(table, W, b).reshape(-1)
    flat = _gather_fn(B * L)(tv, batch.T.reshape(-1))
    return flat.reshape(L, B).T.reshape(B, L, 1)


# speedup vs baseline: 51.7287x; 1.0042x over previous
import functools

import jax
import jax.numpy as jnp
from jax import lax
from jax.experimental import pallas as pl
from jax.experimental.pallas import tpu as pltpu
from jax.experimental.pallas import tpu_sc as plsc

_VOCAB = 1_000_000
_EMB = 32
def _proj_body(tt_ref, w_ref, b_ref, out_ref):
    y = jnp.sum(tt_ref[...] * w_ref[...], axis=0, keepdims=True) + b_ref[0, 0]
    out_ref[...] = y.reshape(1, 1, _PROJ_COLS)


_PROJ_COLS = 65_536
_PROJ_GRID = 16


def _project(table, W, b):
    return pl.pallas_call(
        _proj_body,
        grid=(_PROJ_GRID,),
        in_specs=[
            pl.BlockSpec((_EMB, _PROJ_COLS), lambda i: (0, i)),
            pl.BlockSpec((_EMB, 1), lambda i: (0, 0)),
            pl.BlockSpec(memory_space=pltpu.SMEM),
        ],
        out_specs=pl.BlockSpec((1, 1, _PROJ_COLS), lambda i: (i, 0, 0)),
        out_shape=jax.ShapeDtypeStruct((_PROJ_GRID, 1, _PROJ_COLS), jnp.float32),
    )(table.T, W, b.reshape(1, 1))


@functools.cache
def _gather_fn(total):
    info = plsc.get_sparse_core_info()
    nc, ns = info.num_cores, info.num_subcores
    nw = nc * ns
    per_w = total // nw
    assert per_w * nw == total and per_w % 8 == 0
    chunk = 6_400
    nchunk = per_w // chunk
    assert nchunk * chunk == per_w

    mesh = plsc.VectorSubcoreMesh(core_axis_name="c", subcore_axis_name="s")

    @functools.partial(
        pl.kernel,
        out_type=jax.ShapeDtypeStruct((total,), jnp.float32),
        mesh=mesh,
        scratch_types=[
            pltpu.VMEM((chunk,), jnp.int32),
            pltpu.VMEM((chunk,), jnp.int32),
            pltpu.VMEM((chunk,), jnp.float32),
            pltpu.VMEM((chunk,), jnp.float32),
            pltpu.SemaphoreType.DMA,
            pltpu.SemaphoreType.DMA,
            pltpu.SemaphoreType.DMA,
            pltpu.SemaphoreType.DMA,
            pltpu.SemaphoreType.DMA,
            pltpu.SemaphoreType.DMA,
        ],
    )
    def body(tv_hbm, idx_hbm, out_hbm, i0, i1, r0, r1, si0, si1, sg0, sg1, so0, so1):
        idx_v = [i0, i1]
        rows_v = [r0, r1]
        s_i = [si0, si1]
        s_g = [sg0, sg1]
        s_o = [so0, so1]
        wid = lax.axis_index("s") * nc + lax.axis_index("c")
        base = wid * per_w

        def off(c):
            return base + c * chunk

        h_g = [None, None]
        h_o = [None, None]
        h_i = [None, None]
        h_i[0] = pltpu.async_copy(idx_hbm.at[pl.ds(off(0), chunk)], idx_v[0], s_i[0])
        for c in range(nchunk):
            s = c & 1
            if h_o[s] is not None:
                h_o[s].wait()
            h_i[s].wait()
            h_g[s] = pltpu.async_copy(tv_hbm.at[idx_v[s]], rows_v[s], s_g[s])
            if c >= 1:
                h_g[1 - s].wait()
                h_o[1 - s] = pltpu.async_copy(
                    rows_v[1 - s], out_hbm.at[pl.ds(off(c - 1), chunk)], s_o[1 - s]
                )
            if c + 1 < nchunk:
                h_i[1 - s] = pltpu.async_copy(
                    idx_hbm.at[pl.ds(off(c + 1), chunk)], idx_v[1 - s], s_i[1 - s]
                )
        sl = (nchunk - 1) & 1
        h_g[sl].wait()
        h_o[sl] = pltpu.async_copy(
            rows_v[sl], out_hbm.at[pl.ds(off(nchunk - 1), chunk)], s_o[sl]
        )
        h_o[1 - sl].wait()
        h_o[sl].wait()

    return body


def kernel(batch, table, W, b):
    B, L = batch.shape
    tv = _project(table, W, b).reshape(-1)
    flat = _gather_fn(B * L)(tv, batch.T.reshape(-1))
    return flat.reshape(L, B).T.reshape(B, L, 1)
